# baseline (speedup 1.0000x reference)
import functools

import jax
import jax.numpy as jnp
from jax import lax
from jax.experimental import pallas as pl
from jax.experimental.pallas import tpu as pltpu

_K = 100


def _stats_kernel(x_ref, bv_ref, s_ref, npos_ref, *, hw):
    xb = x_ref[0]
    m = jnp.max(xb, axis=1, keepdims=True)
    e = jnp.sum(jnp.exp(xb - m), axis=1, keepdims=True)
    s = jnp.sum(xb, axis=1, keepdims=True)
    npos = jnp.sum((xb > 0.0).astype(jnp.float32), axis=1, keepdims=True)
    bv_ref[0] = m + jnp.log(e) - s * (1.0 / hw)
    s_ref[0] = s
    npos_ref[0] = npos


def _select_kernel(bv_ref, s_ref, npos_ref, y_ref, loss_ref, np_ref, *, b, cpad):
    y_col = y_ref[...]
    lane = lax.broadcasted_iota(jnp.int32, (b, cpad), 1)
    is_y = lane == y_col

    s = s_ref[...] + 0.0
    s = jnp.where(is_y, -jnp.inf, s)

    bits = pltpu.bitcast(s, jnp.int32)
    key = jnp.where(bits < 0, bits ^ jnp.int32(0x7FFFFFFF), bits)

    lo0 = jnp.full((b, 1), jnp.int32(-(2**31)), jnp.int32)
    hi0 = jnp.full((b, 1), jnp.int32(2**31 - 1), jnp.int32)

    def body(_, carry):
        lo, hi = carry
        mid = (lo >> 1) + (hi >> 1) + (lo & hi & 1)
        cnt = jnp.sum((key >= mid).astype(jnp.int32), axis=1, keepdims=True)
        ok = cnt >= _K
        return jnp.where(ok, mid, lo), jnp.where(ok, hi, mid)

    lo, _ = lax.fori_loop(0, 32, body, (lo0, hi0))
    thr = lo

    gt = key > thr
    eq = key == thr
    need = (_K - jnp.sum(gt.astype(jnp.int32), axis=1, keepdims=True)).astype(
        jnp.float32
    )
    eq_f = eq.astype(jnp.float32)
    r = lax.broadcasted_iota(jnp.int32, (cpad, cpad), 0)
    c = lax.broadcasted_iota(jnp.int32, (cpad, cpad), 1)
    tri = (r < c).astype(jnp.float32)
    rank = lax.dot_general(
        eq_f, tri, (((1,), (0,)), ((), ())), preferred_element_type=jnp.float32
    )
    sel = gt | (eq & (rank < need))

    bv = bv_ref[...]
    loss_ref[...] = jnp.sum(jnp.where(sel, bv, 0.0), keepdims=True).reshape(1, 1) * (
        1.0 / b
    )

    npos = npos_ref[...]
    row_npos = jnp.sum(npos, axis=1, keepdims=True) - jnp.sum(
        jnp.where(is_y, npos, 0.0), axis=1, keepdims=True
    )
    np_ref[...] = jnp.sum(row_npos.astype(jnp.int32), keepdims=True).reshape(1, 1)


@jax.jit
def kernel(x, y):
    B, C, H, W = x.shape
    HW = H * W
    x_flat = x.reshape(B, C, HW)

    bv, s, npos = pl.pallas_call(
        functools.partial(_stats_kernel, hw=HW),
        grid=(B,),
        in_specs=[pl.BlockSpec((1, C, HW), lambda i: (i, 0, 0))],
        out_specs=[
            pl.BlockSpec((1, C, 1), lambda i: (i, 0, 0)),
            pl.BlockSpec((1, C, 1), lambda i: (i, 0, 0)),
            pl.BlockSpec((1, C, 1), lambda i: (i, 0, 0)),
        ],
        out_shape=[
            jax.ShapeDtypeStruct((B, C, 1), jnp.float32),
            jax.ShapeDtypeStruct((B, C, 1), jnp.float32),
            jax.ShapeDtypeStruct((B, C, 1), jnp.float32),
        ],
    )(x_flat)

    CPAD = 1024
    pad = CPAD - C
    bv2 = jnp.pad(bv[..., 0], ((0, 0), (0, pad)))
    s2 = jnp.pad(s[..., 0], ((0, 0), (0, pad)), constant_values=-jnp.inf)
    npos2 = jnp.pad(npos[..., 0], ((0, 0), (0, pad)))
    y2 = y.astype(jnp.int32).reshape(B, 1)

    loss, num_posi = pl.pallas_call(
        functools.partial(_select_kernel, b=B, cpad=CPAD),
        in_specs=[
            pl.BlockSpec((B, CPAD), lambda: (0, 0)),
            pl.BlockSpec((B, CPAD), lambda: (0, 0)),
            pl.BlockSpec((B, CPAD), lambda: (0, 0)),
            pl.BlockSpec((B, 1), lambda: (0, 0)),
        ],
        out_specs=[
            pl.BlockSpec((1, 1), lambda: (0, 0)),
            pl.BlockSpec((1, 1), lambda: (0, 0)),
        ],
        out_shape=[
            jax.ShapeDtypeStruct((1, 1), jnp.float32),
            jax.ShapeDtypeStruct((1, 1), jnp.int32),
        ],
    )(bv2, s2, npos2, y2)

    return (loss[0, 0], num_posi[0, 0])


# speedup vs baseline: 7.1240x; 7.1240x over previous
import functools

import jax
import jax.numpy as jnp
from jax import lax
from jax.experimental import pallas as pl
from jax.experimental.pallas import tpu as pltpu

_K = 100


def _stats_kernel(x_ref, bv_ref, s_ref, npos_ref, *, hw):
    xb = x_ref[...]
    m = jnp.max(xb, axis=0)
    e = jnp.sum(jnp.exp(xb - m[None]), axis=0)
    s = jnp.sum(xb, axis=0)
    npos = jnp.sum((xb > 0.0).astype(jnp.float32), axis=0)
    bv_ref[...] = m + jnp.log(e) - s * (1.0 / hw)
    s_ref[...] = s
    npos_ref[...] = npos


def _select_kernel(bv_ref, s_ref, npos_ref, y_ref, loss_ref, np_ref, *, b, c):
    y_row = y_ref[...]
    row = lax.broadcasted_iota(jnp.int32, (c, b), 0)
    is_y = row == y_row

    s = s_ref[...] + 0.0
    s = jnp.where(is_y, -jnp.inf, s)

    bits = pltpu.bitcast(s, jnp.int32)
    key = jnp.where(bits < 0, bits ^ jnp.int32(0x7FFFFFFF), bits)

    lo0 = jnp.full((1, b), jnp.int32(-(2**31)), jnp.int32)
    hi0 = jnp.full((1, b), jnp.int32(2**31 - 1), jnp.int32)

    def body(_, carry):
        lo, hi = carry
        mid = (lo >> 1) + (hi >> 1) + (lo & hi & 1)
        cnt = jnp.sum((key >= mid).astype(jnp.int32), axis=0, keepdims=True)
        ok = cnt >= _K
        return jnp.where(ok, mid, lo), jnp.where(ok, hi, mid)

    lo, _ = lax.fori_loop(0, 32, body, (lo0, hi0))
    thr = lo

    gt = key > thr
    eq = key == thr
    need = (_K - jnp.sum(gt.astype(jnp.int32), axis=0, keepdims=True)).astype(
        jnp.float32
    )
    eq_f = eq.astype(jnp.float32)
    r = lax.broadcasted_iota(jnp.int32, (c, c), 0)
    cc = lax.broadcasted_iota(jnp.int32, (c, c), 1)
    tri = (cc < r).astype(jnp.float32)
    rank = lax.dot_general(
        tri, eq_f, (((1,), (0,)), ((), ())), preferred_element_type=jnp.float32
    )
    sel = gt | (eq & (rank < need))

    bv = bv_ref[...]
    loss_ref[...] = jnp.sum(jnp.where(sel, bv, 0.0), keepdims=True).reshape(1, 1) * (
        1.0 / b
    )

    npos = npos_ref[...]
    col_npos = jnp.sum(npos, axis=0, keepdims=True) - jnp.sum(
        jnp.where(is_y, npos, 0.0), axis=0, keepdims=True
    )
    np_ref[...] = jnp.sum(col_npos.astype(jnp.int32), keepdims=True).reshape(1, 1)


@jax.jit
def kernel(x, y):
    B, C, H, W = x.shape
    HW = H * W
    x_t = jnp.transpose(x, (2, 3, 1, 0)).reshape(HW, C, B)

    CB = 40
    bv, s, npos = pl.pallas_call(
        functools.partial(_stats_kernel, hw=HW),
        grid=(C // CB,),
        in_specs=[pl.BlockSpec((HW, CB, B), lambda j: (0, j, 0))],
        out_specs=[
            pl.BlockSpec((CB, B), lambda j: (j, 0)),
            pl.BlockSpec((CB, B), lambda j: (j, 0)),
            pl.BlockSpec((CB, B), lambda j: (j, 0)),
        ],
        out_shape=[
            jax.ShapeDtypeStruct((C, B), jnp.float32),
            jax.ShapeDtypeStruct((C, B), jnp.float32),
            jax.ShapeDtypeStruct((C, B), jnp.float32),
        ],
    )(x_t)

    y2 = y.astype(jnp.int32).reshape(1, B)

    loss, num_posi = pl.pallas_call(
        functools.partial(_select_kernel, b=B, c=C),
        in_specs=[
            pl.BlockSpec((C, B), lambda: (0, 0)),
            pl.BlockSpec((C, B), lambda: (0, 0)),
            pl.BlockSpec((C, B), lambda: (0, 0)),
            pl.BlockSpec((1, B), lambda: (0, 0)),
        ],
        out_specs=[
            pl.BlockSpec((1, 1), lambda: (0, 0)),
            pl.BlockSpec((1, 1), lambda: (0, 0)),
        ],
        out_shape=[
            jax.ShapeDtypeStruct((1, 1), jnp.float32),
            jax.ShapeDtypeStruct((1, 1), jnp.int32),
        ],
    )(bv, s, npos, y2)

    return (loss[0, 0], num_posi[0, 0])


# speedup vs baseline: 8.1879x; 1.1494x over previous
import functools

import jax
import jax.numpy as jnp
from jax import lax
from jax.experimental import pallas as pl
from jax.experimental.pallas import tpu as pltpu

_K = 100


def _stats_kernel(x_ref, bv_ref, s_ref, npos_ref, *, hw):
    xb = x_ref[...]
    m = jnp.max(xb, axis=0)
    e = jnp.sum(jnp.exp(xb - m[None]), axis=0)
    s = jnp.sum(xb, axis=0)
    npos = jnp.sum((xb > 0.0).astype(jnp.float32), axis=0)
    bv_ref[...] = m + jnp.log(e) - s * (1.0 / hw)
    s_ref[...] = s
    npos_ref[...] = npos


def _select_kernel(bv_ref, s_ref, npos_ref, y_ref, loss_ref, np_ref, *, b, c):
    y_row = y_ref[...]
    row = lax.broadcasted_iota(jnp.int32, (c, b), 0)
    is_y = row == y_row

    s = s_ref[...] + 0.0
    s = jnp.where(is_y, -jnp.inf, s)

    bits = pltpu.bitcast(s, jnp.int32)
    key = jnp.where(bits < 0, bits ^ jnp.int32(0x7FFFFFFF), bits)

    lo0 = jnp.full((1, b), jnp.int32(-(2**31)), jnp.int32)
    hi0 = jnp.full((1, b), jnp.int32(2**31 - 1), jnp.int32)

    def body(_, carry):
        lo, hi = carry
        mid = (lo >> 1) + (hi >> 1) + (lo & hi & 1)
        cnt = jnp.sum((key >= mid).astype(jnp.int32), axis=0, keepdims=True)
        ok = cnt >= _K
        return jnp.where(ok, mid, lo), jnp.where(ok, hi, mid)

    lo, _ = lax.fori_loop(0, 32, body, (lo0, hi0))
    thr = lo

    gt = key > thr
    eq = key == thr
    need = (_K - jnp.sum(gt.astype(jnp.int32), axis=0, keepdims=True)).astype(
        jnp.float32
    )
    eq_f = eq.astype(jnp.float32)
    r = lax.broadcasted_iota(jnp.int32, (c, c), 0)
    cc = lax.broadcasted_iota(jnp.int32, (c, c), 1)
    tri = (cc < r).astype(jnp.float32)
    rank = lax.dot_general(
        tri, eq_f, (((1,), (0,)), ((), ())), preferred_element_type=jnp.float32
    )
    sel = gt | (eq & (rank < need))

    bv = bv_ref[...]
    loss_ref[...] = jnp.sum(jnp.where(sel, bv, 0.0), keepdims=True).reshape(1, 1) * (
        1.0 / b
    )

    npos = npos_ref[...]
    col_npos = jnp.sum(npos, axis=0, keepdims=True) - jnp.sum(
        jnp.where(is_y, npos, 0.0), axis=0, keepdims=True
    )
    np_ref[...] = jnp.sum(col_npos.astype(jnp.int32), keepdims=True).reshape(1, 1)


@jax.jit
def kernel(x, y):
    B, C, H, W = x.shape
    HW = H * W
    x_t = jnp.transpose(x, (2, 3, 1, 0)).reshape(HW, C, B)

    CB = 200
    bv, s, npos = pl.pallas_call(
        functools.partial(_stats_kernel, hw=HW),
        grid=(C // CB,),
        in_specs=[pl.BlockSpec((HW, CB, B), lambda j: (0, j, 0))],
        out_specs=[
            pl.BlockSpec((CB, B), lambda j: (j, 0)),
            pl.BlockSpec((CB, B), lambda j: (j, 0)),
            pl.BlockSpec((CB, B), lambda j: (j, 0)),
        ],
        out_shape=[
            jax.ShapeDtypeStruct((C, B), jnp.float32),
            jax.ShapeDtypeStruct((C, B), jnp.float32),
            jax.ShapeDtypeStruct((C, B), jnp.float32),
        ],
    )(x_t)

    y2 = y.astype(jnp.int32).reshape(1, B)

    loss, num_posi = pl.pallas_call(
        functools.partial(_select_kernel, b=B, c=C),
        in_specs=[
            pl.BlockSpec((C, B), lambda: (0, 0)),
            pl.BlockSpec((C, B), lambda: (0, 0)),
            pl.BlockSpec((C, B), lambda: (0, 0)),
            pl.BlockSpec((1, B), lambda: (0, 0)),
        ],
        out_specs=[
            pl.BlockSpec((1, 1), lambda: (0, 0)),
            pl.BlockSpec((1, 1), lambda: (0, 0)),
        ],
        out_shape=[
            jax.ShapeDtypeStruct((1, 1), jnp.float32),
            jax.ShapeDtypeStruct((1, 1), jnp.int32),
        ],
    )(bv, s, npos, y2)

    return (loss[0, 0], num_posi[0, 0])


# speedup vs baseline: 8.1964x; 1.0010x over previous
import functools

import jax
import jax.numpy as jnp
from jax import lax
from jax.experimental import pallas as pl
from jax.experimental.pallas import tpu as pltpu

_K = 100


def _stats_kernel(x_ref, bv_ref, s_ref, npos_ref, *, hw):
    xb = x_ref[...]
    m = jnp.max(xb, axis=0)
    e = jnp.sum(jnp.exp(xb - m[None]), axis=0)
    s = jnp.sum(xb, axis=0)
    npos = jnp.sum((xb > 0.0).astype(jnp.float32), axis=0)
    bv_ref[...] = m + jnp.log(e) - s * (1.0 / hw)
    s_ref[...] = s
    npos_ref[...] = npos


def _select_kernel(bv_ref, s_ref, npos_ref, y_ref, loss_ref, np_ref, *, b, c):
    y_row = y_ref[...]
    row = lax.broadcasted_iota(jnp.int32, (c, b), 0)
    is_y = row == y_row

    s = s_ref[...] + 0.0
    s = jnp.where(is_y, -jnp.inf, s)

    bits = pltpu.bitcast(s, jnp.int32)
    key = jnp.where(bits < 0, bits ^ jnp.int32(0x7FFFFFFF), bits)

    lo0 = jnp.full((1, b), jnp.int32(-(2**31)), jnp.int32)
    hi0 = jnp.full((1, b), jnp.int32(2**31 - 1), jnp.int32)

    def body(_, carry):
        lo, hi = carry
        mid = (lo >> 1) + (hi >> 1) + (lo & hi & 1)
        cnt = jnp.sum((key >= mid).astype(jnp.int32), axis=0, keepdims=True)
        ok = cnt >= _K
        return jnp.where(ok, mid, lo), jnp.where(ok, hi, mid)

    lo, _ = lax.fori_loop(0, 32, body, (lo0, hi0))
    thr = lo

    gt = key > thr
    eq = key == thr
    need = (_K - jnp.sum(gt.astype(jnp.int32), axis=0, keepdims=True)).astype(
        jnp.float32
    )
    eq_f = eq.astype(jnp.float32)
    r = lax.broadcasted_iota(jnp.int32, (c, c), 0)
    cc = lax.broadcasted_iota(jnp.int32, (c, c), 1)
    tri = (cc < r).astype(jnp.float32)
    rank = lax.dot_general(
        tri, eq_f, (((1,), (0,)), ((), ())), preferred_element_type=jnp.float32
    )
    sel = gt | (eq & (rank < need))

    bv = bv_ref[...]
    loss_ref[...] = jnp.sum(jnp.where(sel, bv, 0.0), keepdims=True).reshape(1, 1) * (
        1.0 / b
    )

    npos = npos_ref[...]
    col_npos = jnp.sum(npos, axis=0, keepdims=True) - jnp.sum(
        jnp.where(is_y, npos, 0.0), axis=0, keepdims=True
    )
    np_ref[...] = jnp.sum(col_npos.astype(jnp.int32), keepdims=True).reshape(1, 1)


@jax.jit
def kernel(x, y):
    B, C, H, W = x.shape
    HW = H * W
    x_t = jnp.transpose(x, (2, 3, 1, 0)).reshape(HW, C, B)

    CB = 200
    bv, s, npos = pl.pallas_call(
        functools.partial(_stats_kernel, hw=HW),
        grid=(C // CB,),
        in_specs=[pl.BlockSpec((HW, CB, B), lambda j: (0, j, 0))],
        out_specs=[
            pl.BlockSpec((CB, B), lambda j: (j, 0)),
            pl.BlockSpec((CB, B), lambda j: (j, 0)),
            pl.BlockSpec((CB, B), lambda j: (j, 0)),
        ],
        out_shape=[
            jax.ShapeDtypeStruct((C, B), jnp.float32),
            jax.ShapeDtypeStruct((C, B), jnp.float32),
            jax.ShapeDtypeStruct((C, B), jnp.float32),
        ],
        compiler_params=pltpu.CompilerParams(
            dimension_semantics=("parallel",),
        ),
    )(x_t)

    y2 = y.astype(jnp.int32).reshape(1, B)

    loss, num_posi = pl.pallas_call(
        functools.partial(_select_kernel, b=B, c=C),
        in_specs=[
            pl.BlockSpec((C, B), lambda: (0, 0)),
            pl.BlockSpec((C, B), lambda: (0, 0)),
            pl.BlockSpec((C, B), lambda: (0, 0)),
            pl.BlockSpec((1, B), lambda: (0, 0)),
        ],
        out_specs=[
            pl.BlockSpec((1, 1), lambda: (0, 0)),
            pl.BlockSpec((1, 1), lambda: (0, 0)),
        ],
        out_shape=[
            jax.ShapeDtypeStruct((1, 1), jnp.float32),
            jax.ShapeDtypeStruct((1, 1), jnp.int32),
        ],
    )(bv, s, npos, y2)

    return (loss[0, 0], num_posi[0, 0])
